# baseline (speedup 1.0000x reference)
import jax
import jax.numpy as jnp
from jax.experimental import pallas as pl

_EMB = 1024
_DFF = 2048
_E = 8
_W_IMPORTANCE = 0.01


def _gate_kernel(q_ref, wv_ref, bv_ref, wo_ref, bo_ref, gw_ref, gb_ref,
                 prob_ref, ct_ref, imp_ref):
    t = pl.program_id(0)
    v = jnp.dot(q_ref[...], wv_ref[...], preferred_element_type=jnp.float32)
    v = v + bv_ref[...]
    att = jnp.dot(v, wo_ref[...], preferred_element_type=jnp.float32)
    att = att + bo_ref[...]
    logits = jnp.dot(att, gw_ref[...], preferred_element_type=jnp.float32)
    logits = logits + gb_ref[...]
    lmax = jnp.max(logits, axis=1, keepdims=True)
    ex = jnp.exp(logits - lmax)
    p = ex / jnp.sum(ex, axis=1, keepdims=True)
    prob_ref[...] = p

    iota = jax.lax.broadcasted_iota(jnp.int32, p.shape, 1)
    m1 = jnp.max(p, axis=1, keepdims=True)
    i1 = jnp.min(jnp.where(p == m1, iota, _E), axis=1, keepdims=True)
    oh1 = iota == i1
    pm = jnp.where(oh1, -jnp.inf, p)
    m2 = jnp.max(pm, axis=1, keepdims=True)
    i2 = jnp.min(jnp.where(pm == m2, iota, _E), axis=1, keepdims=True)
    oh2 = iota == i2
    e21 = jnp.exp(m2 - m1)
    w1 = 1.0 / (1.0 + e21)
    w2 = e21 / (1.0 + e21)
    c = jnp.where(oh1, w1, 0.0) + jnp.where(oh2, w2, 0.0)
    ct_ref[...] = c

    @pl.when(t == 0)
    def _():
        imp_ref[...] = jnp.zeros_like(imp_ref)

    imp_ref[...] += jnp.sum(p, axis=0, keepdims=True)


def _moe_kernel(x_ref, w1_ref, b1_ref, w2_ref, b2_ref, c_ref, y_ref):
    e = pl.program_id(0)
    j = pl.program_id(1)
    h = jnp.dot(x_ref[...], w1_ref[0], preferred_element_type=jnp.float32)
    h = jnp.maximum(h + b1_ref[0], 0.0).astype(jnp.bfloat16)
    part = jnp.dot(h, w2_ref[0], preferred_element_type=jnp.float32)
    cb = c_ref[0]

    @pl.when((e == 0) & (j == 0))
    def _():
        y_ref[...] = jnp.zeros_like(y_ref)

    y_ref[...] += part * cb

    @pl.when(j == 0)
    def _():
        y_ref[...] += cb * b2_ref[0]


def kernel(x, q, Wq, bq, Wk, bk, Wv, bv, Wo, bo, gate_W, gate_b, W1, b1, W2, b2):
    x_shape = x.shape
    xf = x.reshape(-1, x_shape[-1])
    N, d = xf.shape
    TM = 1024
    T = N // TM

    gate_prob, c, imp = pl.pallas_call(
        _gate_kernel,
        grid=(T,),
        in_specs=[
            pl.BlockSpec((TM, d), lambda t: (t, 0)),
            pl.BlockSpec((d, d), lambda t: (0, 0)),
            pl.BlockSpec((1, d), lambda t: (0, 0)),
            pl.BlockSpec((d, d), lambda t: (0, 0)),
            pl.BlockSpec((1, d), lambda t: (0, 0)),
            pl.BlockSpec((d, _E), lambda t: (0, 0)),
            pl.BlockSpec((1, _E), lambda t: (0, 0)),
        ],
        out_specs=[
            pl.BlockSpec((TM, _E), lambda t: (t, 0)),
            pl.BlockSpec((TM, _E), lambda t: (t, 0)),
            pl.BlockSpec((1, _E), lambda t: (0, 0)),
        ],
        out_shape=[
            jax.ShapeDtypeStruct((N, _E), jnp.float32),
            jax.ShapeDtypeStruct((N, _E), jnp.float32),
            jax.ShapeDtypeStruct((1, _E), jnp.float32),
        ],
    )(q, Wv, bv.reshape(1, d), Wo, bo.reshape(1, d),
      gate_W, gate_b.reshape(1, _E))

    cT = c.T.reshape(_E, N, 1)

    FJ = 512
    J = _DFF // FJ
    xb = xf.astype(jnp.bfloat16)
    w1b = W1.astype(jnp.bfloat16)
    w2b = W2.astype(jnp.bfloat16)
    b1r = b1.reshape(_E, 1, _DFF)
    b2r = b2.reshape(_E, 1, d)

    y = pl.pallas_call(
        _moe_kernel,
        grid=(_E, J),
        in_specs=[
            pl.BlockSpec((N, d), lambda e, j: (0, 0)),
            pl.BlockSpec((1, d, FJ), lambda e, j: (e, 0, j)),
            pl.BlockSpec((1, 1, FJ), lambda e, j: (e, 0, j)),
            pl.BlockSpec((1, FJ, d), lambda e, j: (e, j, 0)),
            pl.BlockSpec((1, 1, d), lambda e, j: (e, 0, 0)),
            pl.BlockSpec((1, N, 1), lambda e, j: (e, 0, 0)),
        ],
        out_specs=pl.BlockSpec((N, d), lambda e, j: (0, 0)),
        out_shape=jax.ShapeDtypeStruct((N, d), jnp.float32),
    )(xb, w1b, b1r, w2b, b2r, cT)

    importance = imp[0]
    importance_loss = _W_IMPORTANCE * (
        jnp.std(importance, ddof=1) / jnp.mean(importance)) ** 2
    return y.reshape(x_shape), gate_prob, importance_loss
